# baseline (speedup 1.0000x reference)
import jax
import jax.numpy as jnp
from jax.experimental import pallas as pl

_S = 64.0
_M = 0.4

_BLOCK_B = 16


def _body(labels_ref, x_ref, o_ref):
    bb, c = x_ref.shape
    cols = jax.lax.broadcasted_iota(jnp.int32, (bb, c), 1)
    lab = labels_ref[...]
    x = x_ref[...]
    o_ref[...] = (x - jnp.where(cols == lab, _M, 0.0)) * _S


def kernel(logits, norms, labels):
    del norms
    b, c = logits.shape
    labels2d = labels.astype(jnp.int32).reshape(b, 1)
    grid = (b // _BLOCK_B,)
    return pl.pallas_call(
        _body,
        grid=grid,
        in_specs=[
            pl.BlockSpec((_BLOCK_B, 1), lambda i: (i, 0)),
            pl.BlockSpec((_BLOCK_B, c), lambda i: (i, 0)),
        ],
        out_specs=pl.BlockSpec((_BLOCK_B, c), lambda i: (i, 0)),
        out_shape=jax.ShapeDtypeStruct((b, c), jnp.float32),
    )(labels2d, logits)
